# Initial kernel scaffold; baseline (speedup 1.0000x reference)
import jax
import jax.numpy as jnp
from jax.experimental import pallas as pl


def kernel(positive_sample, negative_sample, entity_static_embeddings, entity_dynamic_embeddings, relation_embeddings):
    raise NotImplementedError("write your pallas kernel here")


# speedup vs baseline: 4.0465x; 4.0465x over previous
import functools

import jax
import jax.numpy as jnp
from jax import lax
from jax.experimental import pallas as pl
from jax.experimental.pallas import tpu as pltpu
from jax.experimental.pallas import tpu_sc as plsc

_GAMMA = 12.0
_D = 128
_NEG = 200
_NEGP = 208
_B = 4096
_NC = 2
_NS = 16
_NW = _NC * _NS
_BPW = _B // _NW
_L = 16


def _combine_body(a_ref, b_ref, o_ref):
    o_ref[...] = a_ref[...] + b_ref[...]


def _combine(a, b):
    rows = a.shape[0]
    blk = 2000
    return pl.pallas_call(
        _combine_body,
        grid=(rows // blk,),
        in_specs=[pl.BlockSpec((blk, _D), lambda i: (i, 0))] * 2,
        out_specs=pl.BlockSpec((blk, _D), lambda i: (i, 0)),
        out_shape=jax.ShapeDtypeStruct((rows, _D), jnp.float32),
    )(a, b)


def _score_body(pos0_hbm, pos1_hbm, neg_hbm, comb_hbm, rel_hbm, out_hbm,
                negv, hrv, tbuf, scorev, redv, p0v, p1v, sem0, sem1):
    cid = lax.axis_index("c")
    sid = lax.axis_index("s")
    wid = sid * _NC + cid
    base = wid * _BPW

    pltpu.sync_copy(pos0_hbm.at[pl.ds(base, _BPW)], p0v)
    pltpu.sync_copy(pos1_hbm.at[pl.ds(base, _BPW)], p1v)
    pltpu.async_copy(comb_hbm.at[p0v], hrv, sem0).wait()
    pltpu.async_copy(rel_hbm.at[p1v], tbuf.at[pl.ds(0, _BPW)], sem0).wait()

    def _add_body(r, carry):
        for c in range(_D // _L):
            sl = pl.ds(c * _L, _L)
            hrv[r, sl] = hrv[r, sl] + tbuf[r, sl]
        return carry
    lax.fori_loop(0, _BPW, _add_body, 0)

    pltpu.sync_copy(neg_hbm.at[pl.ds(base, _BPW)], negv)

    lane = lax.iota(jnp.int32, _L)

    def _b_body(b, carry):
        cp0 = pltpu.async_copy(comb_hbm.at[negv.at[b, 0]],
                               tbuf.at[pl.ds(0, _NEGP // 2)], sem0)
        cp1 = pltpu.async_copy(comb_hbm.at[negv.at[b, 1]],
                               tbuf.at[pl.ds(_NEGP // 2, _NEGP // 2)], sem1)
        cp0.wait()
        cp1.wait()

        hch = [hrv[b, pl.ds(c * _L, _L)] for c in range(_D // _L)]

        col_base = lane * _L

        def _g_body(g, carry2):
            for j in range(_L):
                row = g * _L + j
                acc = jnp.abs(tbuf[row, pl.ds(0, _L)] - hch[0])
                for c in range(1, _D // _L):
                    acc = acc + jnp.abs(tbuf[row, pl.ds(c * _L, _L)] - hch[c])
                redv[pl.ds(j * _L, _L)] = acc
            tot = plsc.load_gather(redv, [col_base])
            for c in range(1, _L):
                tot = tot + plsc.load_gather(redv, [col_base + c])
            scorev[b, pl.ds(g * _L, _L)] = _GAMMA - tot
            return carry2
        lax.fori_loop(0, _NEGP // _L, _g_body, 0)
        return carry
    lax.fori_loop(0, _BPW, _b_body, 0)

    pltpu.sync_copy(scorev, out_hbm.at[pl.ds(base, _BPW)])


@functools.partial(
    pl.kernel,
    out_type=jax.ShapeDtypeStruct((_B, _NEGP), jnp.float32),
    mesh=plsc.VectorSubcoreMesh(core_axis_name="c", subcore_axis_name="s",
                                num_cores=_NC, num_subcores=_NS),
    compiler_params=pltpu.CompilerParams(needs_layout_passes=False),
    scratch_types=[
        pltpu.VMEM((_BPW, 2, _NEGP // 2), jnp.int32),
        pltpu.VMEM((_BPW, _D), jnp.float32),
        pltpu.VMEM((_NEGP, _D), jnp.float32),
        pltpu.VMEM((_BPW, _NEGP), jnp.float32),
        pltpu.VMEM((_L * _L,), jnp.float32),
        pltpu.VMEM((_BPW,), jnp.int32),
        pltpu.VMEM((_BPW,), jnp.int32),
        pltpu.SemaphoreType.DMA,
        pltpu.SemaphoreType.DMA,
    ],
)
def _score(pos0_hbm, pos1_hbm, neg_hbm, comb_hbm, rel_hbm, out_hbm,
           negv, hrv, tbuf, scorev, redv, p0v, p1v, sem0, sem1):
    _score_body(pos0_hbm, pos1_hbm, neg_hbm, comb_hbm, rel_hbm, out_hbm,
                negv, hrv, tbuf, scorev, redv, p0v, p1v, sem0, sem1)


def kernel(positive_sample, negative_sample, entity_static_embeddings,
           entity_dynamic_embeddings, relation_embeddings):
    pos0 = positive_sample[:, 0].astype(jnp.int32)
    pos1 = positive_sample[:, 1].astype(jnp.int32)
    neg = jnp.pad(negative_sample.astype(jnp.int32),
                  ((0, 0), (0, _NEGP - _NEG)))
    neg3 = neg.reshape(_B, 2, _NEGP // 2)
    comb = _combine(entity_static_embeddings, entity_dynamic_embeddings)
    out = _score(pos0, pos1, neg3, comb, relation_embeddings)
    return out[:, :_NEG]
